# baseline (speedup 1.0000x reference)
import functools

import jax
import jax.numpy as jnp
from jax import lax
from jax.experimental import pallas as pl
from jax.experimental.pallas import tpu as pltpu
from jax.experimental.pallas import tpu_sc as plsc

T, H, F, E, K = 2048, 1024, 512, 8, 2
P = T * K
EPAD = 128
TB = 128
NB = P // TB + E
NP = NB * TB

NC, NS, L = 2, 16, 16
NW = NC * NS

HW = H // 2
HQ = HW // NC
HR = HQ // 2
SLOTS_S = NP // NS
G_CH = 80
G_NCH = SLOTS_S // G_CH
TOK_W = T // NW
T_CH = 32


def _router_body(x_ref, gt_ref, iw_ref, ww_ref):
    logits = jnp.dot(x_ref[...], gt_ref[...], preferred_element_type=jnp.float32)
    col = lax.broadcasted_iota(jnp.int32, (T, EPAD), 1)
    valid = col < E
    lg = jnp.where(valid, logits, -1e30)
    m = jnp.max(lg, axis=1, keepdims=True)
    z = jnp.exp(lg - m)
    prob = z / jnp.sum(z, axis=1, keepdims=True)
    prob = jnp.where(valid, prob, -1.0)
    m1 = jnp.max(prob, axis=1, keepdims=True)
    i1 = jnp.min(jnp.where(prob == m1, col, EPAD), axis=1, keepdims=True)
    p2 = jnp.where(col == i1, -1.0, prob)
    m2 = jnp.max(p2, axis=1, keepdims=True)
    i2 = jnp.min(jnp.where(p2 == m2, col, EPAD), axis=1, keepdims=True)
    denom = m1 + m2 + 1e-20
    w1 = m1 / denom
    w2 = m2 / denom
    iw_ref[...] = jnp.where(col == 0, i1, jnp.where(col == 1, i2, 0))
    ww_ref[...] = jnp.where(col == 0, w1, jnp.where(col == 1, w2, 0.0))


def _expert_body(bexp_ref, bval_ref, xs_ref, gw_ref, uw_ref, dw_ref, sw_ref, out_ref):
    b = pl.program_id(0)

    @pl.when(bval_ref[b] == 1)
    def _():
        xi = xs_ref[...]
        lo = lax.bitcast_convert_type(
            (xi & 0xFFFF).astype(jnp.uint16), jnp.bfloat16)
        hi = lax.bitcast_convert_type(
            lax.shift_right_logical(xi, 16).astype(jnp.uint16),
            jnp.bfloat16)
        g = (jnp.dot(lo, gw_ref[0, :HW], preferred_element_type=jnp.float32)
             + jnp.dot(hi, gw_ref[0, HW:], preferred_element_type=jnp.float32))
        u = (jnp.dot(lo, uw_ref[0, :HW], preferred_element_type=jnp.float32)
             + jnp.dot(hi, uw_ref[0, HW:], preferred_element_type=jnp.float32))
        a = (g / (1.0 + jnp.exp(-g))) * u
        o = jnp.dot(a.astype(jnp.bfloat16), dw_ref[0],
                    preferred_element_type=jnp.float32)
        out_ref[...] = o * sw_ref[0, 0, :][:, None]


@functools.cache
def _make_sc_gather():
    mesh = plsc.VectorSubcoreMesh(core_axis_name="c", subcore_axis_name="s")

    @functools.partial(
        pl.kernel,
        mesh=mesh,
        out_type=jax.ShapeDtypeStruct((NP, HW), jnp.int32),
        scratch_types=[
            pltpu.VMEM((SLOTS_S,), jnp.int32),
            pltpu.VMEM((G_CH, HR), jnp.int32),
            pltpu.VMEM((G_CH, HR), jnp.int32),
            pltpu.VMEM_SHARED((T, HR), jnp.int32),
            pltpu.SemaphoreType.DMA,
            pltpu.SemaphoreType.DMA,
            pltpu.SemaphoreType.DMA,
            pltpu.SemaphoreType.DMA,
        ],
    )
    def _sc_gather(x_hbm, tok_hbm, xs_hbm, tok_v, rows0, rows1, xsh,
                   gs0, gs1, ws0, ws1):
        sid = lax.axis_index("s")
        cid = lax.axis_index("c")
        base = sid * SLOTS_S
        pltpu.sync_copy(tok_hbm.at[pl.ds(base, SLOTS_S)], tok_v)
        rows = (rows0, rows1)
        gsem = (gs0, gs1)
        wsem = (ws0, ws1)
        for r in range(2):
            coff = cid * HQ + r * HR
            if r > 0:
                plsc.subcore_barrier()

            @pl.when(sid == 0)
            def _(coff=coff):
                pltpu.sync_copy(x_hbm.at[:, pl.ds(coff, HR)], xsh)

            plsc.subcore_barrier()
            gc = [None] * G_NCH
            wc = [None] * G_NCH
            gc[0] = pltpu.async_copy(
                xsh.at[tok_v.at[pl.ds(0, G_CH)]], rows[0], gsem[0])
            for c in range(G_NCH):
                b = c % 2
                nb = (c + 1) % 2
                if c + 1 < G_NCH:
                    if c >= 1:
                        wc[c - 1].wait()
                    gc[c + 1] = pltpu.async_copy(
                        xsh.at[tok_v.at[pl.ds((c + 1) * G_CH, G_CH)]],
                        rows[nb], gsem[nb])
                gc[c].wait()
                wc[c] = pltpu.async_copy(
                    rows[b],
                    xs_hbm.at[pl.ds(base + c * G_CH, G_CH),
                              pl.ds(coff, HR)],
                    wsem[b])
            for c in range(max(G_NCH - 2, 0), G_NCH):
                wc[c].wait()

    return _sc_gather


@functools.cache
def _make_sc_combine():
    mesh = plsc.VectorSubcoreMesh(core_axis_name="c", subcore_axis_name="s")

    @functools.partial(
        pl.kernel,
        mesh=mesh,
        out_type=jax.ShapeDtypeStruct((T, H), jnp.float32),
        scratch_types=[
            pltpu.VMEM((2 * T_CH,), jnp.int32),
            pltpu.VMEM((2 * T_CH, H), jnp.float32),
            pltpu.VMEM((T_CH, H), jnp.float32),
            pltpu.SemaphoreType.DMA,
        ],
    )
    def _sc_combine(ysw_hbm, pos_hbm, out_hbm, idx_v, rows_v, out_v, sem):
        wid = lax.axis_index("s") * NC + lax.axis_index("c")
        for c in range(TOK_W // T_CH):
            tbase = wid * TOK_W + c * T_CH
            pltpu.sync_copy(pos_hbm.at[pl.ds(2 * tbase, 2 * T_CH)], idx_v)
            pltpu.async_copy(ysw_hbm.at[idx_v], rows_v, sem).wait()

            def body(i, carry):
                for v in range(H // L):
                    sl = pl.ds(v * L, L)
                    out_v[i, sl] = rows_v[2 * i, sl] + rows_v[2 * i + 1, sl]
                return carry

            lax.fori_loop(0, T_CH, body, 0)
            pltpu.sync_copy(out_v, out_hbm.at[pl.ds(tbase, T_CH)])

    return _sc_combine


def kernel(x, gate_tensor, gate_w, up_w, down_w):
    gt_pad = jnp.pad(gate_tensor, ((0, 0), (0, EPAD - E)))
    iw, ww = pl.pallas_call(
        _router_body,
        out_shape=(
            jax.ShapeDtypeStruct((T, EPAD), jnp.int32),
            jax.ShapeDtypeStruct((T, EPAD), jnp.float32),
        ),
    )(x, gt_pad)
    topk_idx = iw[:, :K]
    flat_w = ww[:, :K].reshape(-1)

    flat_e = topk_idx.reshape(-1)
    onehot = (flat_e[:, None] == jnp.arange(E, dtype=jnp.int32)[None, :]).astype(jnp.int32)
    csum = jnp.cumsum(onehot, axis=0)
    rank = jnp.take_along_axis(csum, flat_e[:, None], axis=1)[:, 0] - 1
    counts = csum[-1]
    nblk = (counts + TB - 1) // TB
    blk_start = jnp.concatenate([jnp.zeros(1, jnp.int32), jnp.cumsum(nblk)[:-1].astype(jnp.int32)])
    tot_blocks = jnp.sum(nblk)
    slot = blk_start[flat_e] * TB + rank
    tok_of_slot = jnp.zeros(NP, jnp.int32).at[slot].set(
        jnp.arange(P, dtype=jnp.int32) // K)
    sortw = jnp.zeros(NP, jnp.float32).at[slot].set(flat_w)
    barange = jnp.arange(NB, dtype=jnp.int32)
    bexp = jnp.sum((barange[:, None] >= blk_start[None, :]).astype(jnp.int32), axis=1) - 1
    bval = (barange < tot_blocks).astype(jnp.int32)

    xu = lax.bitcast_convert_type(x.astype(jnp.bfloat16), jnp.uint16)
    xu = xu.astype(jnp.uint32)
    x_pk = lax.bitcast_convert_type(
        xu[:, :HW] | (xu[:, HW:] << 16), jnp.int32)
    xs_pk = _make_sc_gather()(x_pk, tok_of_slot)

    grid_spec = pltpu.PrefetchScalarGridSpec(
        num_scalar_prefetch=2,
        grid=(NB,),
        in_specs=[
            pl.BlockSpec((TB, HW), lambda b, be, bv: (b, 0)),
            pl.BlockSpec((1, H, F), lambda b, be, bv: (be[b], 0, 0)),
            pl.BlockSpec((1, H, F), lambda b, be, bv: (be[b], 0, 0)),
            pl.BlockSpec((1, F, H), lambda b, be, bv: (be[b], 0, 0)),
            pl.BlockSpec((1, 1, TB), lambda b, be, bv: (b, 0, 0)),
        ],
        out_specs=pl.BlockSpec((TB, H), lambda b, be, bv: (b, 0)),
    )
    ysw = pl.pallas_call(
        _expert_body,
        grid_spec=grid_spec,
        out_shape=jax.ShapeDtypeStruct((NP, H), jnp.float32),
    )(bexp, bval, xs_pk,
      gate_w.astype(jnp.bfloat16), up_w.astype(jnp.bfloat16),
      down_w.astype(jnp.bfloat16), sortw.reshape(NB, 1, TB))

    return _make_sc_combine()(ysw, slot)


# speedup vs baseline: 1.0329x; 1.0329x over previous
import functools

import jax
import jax.numpy as jnp
from jax import lax
from jax.experimental import pallas as pl
from jax.experimental.pallas import tpu as pltpu
from jax.experimental.pallas import tpu_sc as plsc

T, H, F, E, K = 2048, 1024, 512, 8, 2
P = T * K
EPAD = 128
TB = 256
NB = P // TB + E
NP = NB * TB

NC, NS, L = 2, 16, 16
NW = NC * NS

HW = H // 2
HQ = HW // NC
HR = HQ // 2
SLOTS_S = NP // NS
G_CH = 96
G_NCH = SLOTS_S // G_CH
TOK_W = T // NW
T_CH = 32


def _router_body(x_ref, gt_ref, iw_ref, ww_ref):
    logits = jnp.dot(x_ref[...], gt_ref[...], preferred_element_type=jnp.float32)
    col = lax.broadcasted_iota(jnp.int32, (T, EPAD), 1)
    valid = col < E
    lg = jnp.where(valid, logits, -1e30)
    m = jnp.max(lg, axis=1, keepdims=True)
    z = jnp.exp(lg - m)
    prob = z / jnp.sum(z, axis=1, keepdims=True)
    prob = jnp.where(valid, prob, -1.0)
    m1 = jnp.max(prob, axis=1, keepdims=True)
    i1 = jnp.min(jnp.where(prob == m1, col, EPAD), axis=1, keepdims=True)
    p2 = jnp.where(col == i1, -1.0, prob)
    m2 = jnp.max(p2, axis=1, keepdims=True)
    i2 = jnp.min(jnp.where(p2 == m2, col, EPAD), axis=1, keepdims=True)
    denom = m1 + m2 + 1e-20
    w1 = m1 / denom
    w2 = m2 / denom
    iw_ref[...] = jnp.where(col == 0, i1, jnp.where(col == 1, i2, 0))
    ww_ref[...] = jnp.where(col == 0, w1, jnp.where(col == 1, w2, 0.0))


def _expert_body(bexp_ref, bval_ref, xs_ref, gw_ref, uw_ref, dw_ref, sw_ref, out_ref):
    b = pl.program_id(0)

    @pl.when(bval_ref[b] == 1)
    def _():
        xi = xs_ref[...]
        lo = lax.bitcast_convert_type(
            (xi & 0xFFFF).astype(jnp.uint16), jnp.bfloat16)
        hi = lax.bitcast_convert_type(
            lax.shift_right_logical(xi, 16).astype(jnp.uint16),
            jnp.bfloat16)
        g = (jnp.dot(lo, gw_ref[0, :HW], preferred_element_type=jnp.float32)
             + jnp.dot(hi, gw_ref[0, HW:], preferred_element_type=jnp.float32))
        u = (jnp.dot(lo, uw_ref[0, :HW], preferred_element_type=jnp.float32)
             + jnp.dot(hi, uw_ref[0, HW:], preferred_element_type=jnp.float32))
        a = (g / (1.0 + jnp.exp(-g))) * u
        o = jnp.dot(a.astype(jnp.bfloat16), dw_ref[0],
                    preferred_element_type=jnp.float32)
        out_ref[...] = o * sw_ref[0, 0, :][:, None]


@functools.cache
def _make_sc_gather():
    mesh = plsc.VectorSubcoreMesh(core_axis_name="c", subcore_axis_name="s")

    @functools.partial(
        pl.kernel,
        mesh=mesh,
        out_type=jax.ShapeDtypeStruct((NP, HW), jnp.int32),
        scratch_types=[
            pltpu.VMEM((SLOTS_S,), jnp.int32),
            pltpu.VMEM((G_CH, HR), jnp.int32),
            pltpu.VMEM((G_CH, HR), jnp.int32),
            pltpu.VMEM_SHARED((T, HR), jnp.int32),
            pltpu.SemaphoreType.DMA,
            pltpu.SemaphoreType.DMA,
            pltpu.SemaphoreType.DMA,
            pltpu.SemaphoreType.DMA,
        ],
    )
    def _sc_gather(x_hbm, tok_hbm, xs_hbm, tok_v, rows0, rows1, xsh,
                   gs0, gs1, ws0, ws1):
        sid = lax.axis_index("s")
        cid = lax.axis_index("c")
        base = sid * SLOTS_S
        pltpu.sync_copy(tok_hbm.at[pl.ds(base, SLOTS_S)], tok_v)
        rows = (rows0, rows1)
        gsem = (gs0, gs1)
        wsem = (ws0, ws1)
        for r in range(2):
            coff = cid * HQ + r * HR
            if r > 0:
                plsc.subcore_barrier()

            @pl.when(sid == 0)
            def _(coff=coff):
                pltpu.sync_copy(x_hbm.at[:, pl.ds(coff, HR)], xsh)

            plsc.subcore_barrier()
            gc = [None] * G_NCH
            wc = [None] * G_NCH
            gc[0] = pltpu.async_copy(
                xsh.at[tok_v.at[pl.ds(0, G_CH)]], rows[0], gsem[0])
            for c in range(G_NCH):
                b = c % 2
                nb = (c + 1) % 2
                if c + 1 < G_NCH:
                    if c >= 1:
                        wc[c - 1].wait()
                    gc[c + 1] = pltpu.async_copy(
                        xsh.at[tok_v.at[pl.ds((c + 1) * G_CH, G_CH)]],
                        rows[nb], gsem[nb])
                gc[c].wait()
                wc[c] = pltpu.async_copy(
                    rows[b],
                    xs_hbm.at[pl.ds(base + c * G_CH, G_CH),
                              pl.ds(coff, HR)],
                    wsem[b])
            for c in range(max(G_NCH - 2, 0), G_NCH):
                wc[c].wait()

    return _sc_gather


@functools.cache
def _make_sc_combine():
    mesh = plsc.VectorSubcoreMesh(core_axis_name="c", subcore_axis_name="s")

    @functools.partial(
        pl.kernel,
        mesh=mesh,
        out_type=jax.ShapeDtypeStruct((T, H), jnp.float32),
        scratch_types=[
            pltpu.VMEM((2 * T_CH,), jnp.int32),
            pltpu.VMEM((2 * T_CH, H), jnp.float32),
            pltpu.VMEM((T_CH, H), jnp.float32),
            pltpu.SemaphoreType.DMA,
        ],
    )
    def _sc_combine(ysw_hbm, pos_hbm, out_hbm, idx_v, rows_v, out_v, sem):
        wid = lax.axis_index("s") * NC + lax.axis_index("c")
        for c in range(TOK_W // T_CH):
            tbase = wid * TOK_W + c * T_CH
            pltpu.sync_copy(pos_hbm.at[pl.ds(2 * tbase, 2 * T_CH)], idx_v)
            pltpu.async_copy(ysw_hbm.at[idx_v], rows_v, sem).wait()

            def body(i, carry):
                for v in range(H // L):
                    sl = pl.ds(v * L, L)
                    out_v[i, sl] = rows_v[2 * i, sl] + rows_v[2 * i + 1, sl]
                return carry

            lax.fori_loop(0, T_CH, body, 0)
            pltpu.sync_copy(out_v, out_hbm.at[pl.ds(tbase, T_CH)])

    return _sc_combine


def kernel(x, gate_tensor, gate_w, up_w, down_w):
    gt_pad = jnp.pad(gate_tensor, ((0, 0), (0, EPAD - E)))
    iw, ww = pl.pallas_call(
        _router_body,
        out_shape=(
            jax.ShapeDtypeStruct((T, EPAD), jnp.int32),
            jax.ShapeDtypeStruct((T, EPAD), jnp.float32),
        ),
    )(x, gt_pad)
    topk_idx = iw[:, :K]
    flat_w = ww[:, :K].reshape(-1)

    flat_e = topk_idx.reshape(-1)
    onehot = (flat_e[:, None] == jnp.arange(E, dtype=jnp.int32)[None, :]).astype(jnp.int32)
    csum = jnp.cumsum(onehot, axis=0)
    rank = jnp.take_along_axis(csum, flat_e[:, None], axis=1)[:, 0] - 1
    counts = csum[-1]
    nblk = (counts + TB - 1) // TB
    blk_start = jnp.concatenate([jnp.zeros(1, jnp.int32), jnp.cumsum(nblk)[:-1].astype(jnp.int32)])
    tot_blocks = jnp.sum(nblk)
    slot = blk_start[flat_e] * TB + rank
    tok_of_slot = jnp.zeros(NP, jnp.int32).at[slot].set(
        jnp.arange(P, dtype=jnp.int32) // K)
    sortw = jnp.zeros(NP, jnp.float32).at[slot].set(flat_w)
    barange = jnp.arange(NB, dtype=jnp.int32)
    bexp = jnp.sum((barange[:, None] >= blk_start[None, :]).astype(jnp.int32), axis=1) - 1
    bval = (barange < tot_blocks).astype(jnp.int32)

    xu = lax.bitcast_convert_type(x.astype(jnp.bfloat16), jnp.uint16)
    xu = xu.astype(jnp.uint32)
    x_pk = lax.bitcast_convert_type(
        xu[:, :HW] | (xu[:, HW:] << 16), jnp.int32)
    xs_pk = _make_sc_gather()(x_pk, tok_of_slot)

    grid_spec = pltpu.PrefetchScalarGridSpec(
        num_scalar_prefetch=2,
        grid=(NB,),
        in_specs=[
            pl.BlockSpec((TB, HW), lambda b, be, bv: (b, 0)),
            pl.BlockSpec((1, H, F), lambda b, be, bv: (be[b], 0, 0)),
            pl.BlockSpec((1, H, F), lambda b, be, bv: (be[b], 0, 0)),
            pl.BlockSpec((1, F, H), lambda b, be, bv: (be[b], 0, 0)),
            pl.BlockSpec((1, 1, TB), lambda b, be, bv: (b, 0, 0)),
        ],
        out_specs=pl.BlockSpec((TB, H), lambda b, be, bv: (b, 0)),
    )
    ysw = pl.pallas_call(
        _expert_body,
        grid_spec=grid_spec,
        out_shape=jax.ShapeDtypeStruct((NP, H), jnp.float32),
    )(bexp, bval, xs_pk,
      gate_w.astype(jnp.bfloat16), up_w.astype(jnp.bfloat16),
      down_w.astype(jnp.bfloat16), sortw.reshape(NB, 1, TB))

    return _make_sc_combine()(ysw, slot)


# speedup vs baseline: 1.1636x; 1.1266x over previous
import functools

import jax
import jax.numpy as jnp
from jax import lax
from jax.experimental import pallas as pl
from jax.experimental.pallas import tpu as pltpu
from jax.experimental.pallas import tpu_sc as plsc

T, H, F, E, K = 2048, 1024, 512, 8, 2
P = T * K
EPAD = 128
TB = 256
NB = P // TB + E
NP = NB * TB

NC, NS, L = 2, 16, 16
NW = NC * NS

HW = H // 2
HQ = HW // NC
HR = HQ // 2
SLOTS_S = NP // NS
G_CH = 96
G_NCH = SLOTS_S // G_CH
TOK_W = T // NW
T_CH = 32


def _router_body(x_ref, gt_ref, iw_ref, ww_ref):
    logits = jnp.dot(x_ref[...], gt_ref[...], preferred_element_type=jnp.float32)
    col = lax.broadcasted_iota(jnp.int32, (T, EPAD), 1)
    valid = col < E
    lg = jnp.where(valid, logits, -1e30)
    m = jnp.max(lg, axis=1, keepdims=True)
    z = jnp.exp(lg - m)
    prob = z / jnp.sum(z, axis=1, keepdims=True)
    prob = jnp.where(valid, prob, -1.0)
    m1 = jnp.max(prob, axis=1, keepdims=True)
    i1 = jnp.min(jnp.where(prob == m1, col, EPAD), axis=1, keepdims=True)
    p2 = jnp.where(col == i1, -1.0, prob)
    m2 = jnp.max(p2, axis=1, keepdims=True)
    i2 = jnp.min(jnp.where(p2 == m2, col, EPAD), axis=1, keepdims=True)
    denom = m1 + m2 + 1e-20
    w1 = m1 / denom
    w2 = m2 / denom
    iw_ref[...] = jnp.where(col == 0, i1, jnp.where(col == 1, i2, 0))
    ww_ref[...] = jnp.where(col == 0, w1, jnp.where(col == 1, w2, 0.0))


def _expert_body(bexp_ref, bval_ref, xs_ref, gw_ref, uw_ref, dw_ref, sw_ref, out_ref):
    b = pl.program_id(0)

    @pl.when(bval_ref[b] == 1)
    def _():
        xi = xs_ref[...]
        lo = lax.bitcast_convert_type(
            (xi & 0xFFFF).astype(jnp.uint16), jnp.bfloat16)
        hi = lax.bitcast_convert_type(
            lax.shift_right_logical(xi, 16).astype(jnp.uint16),
            jnp.bfloat16)
        gw = gw_ref[0].astype(jnp.bfloat16)
        uw = uw_ref[0].astype(jnp.bfloat16)
        g = (jnp.dot(lo, gw[:HW], preferred_element_type=jnp.float32)
             + jnp.dot(hi, gw[HW:], preferred_element_type=jnp.float32))
        u = (jnp.dot(lo, uw[:HW], preferred_element_type=jnp.float32)
             + jnp.dot(hi, uw[HW:], preferred_element_type=jnp.float32))
        a = (g / (1.0 + jnp.exp(-g))) * u
        o = jnp.dot(a.astype(jnp.bfloat16), dw_ref[0].astype(jnp.bfloat16),
                    preferred_element_type=jnp.float32)
        out_ref[...] = o * sw_ref[0, 0, :][:, None]


@functools.cache
def _make_sc_gather():
    mesh = plsc.VectorSubcoreMesh(core_axis_name="c", subcore_axis_name="s")

    @functools.partial(
        pl.kernel,
        mesh=mesh,
        out_type=jax.ShapeDtypeStruct((NP, HW), jnp.int32),
        scratch_types=[
            pltpu.VMEM((SLOTS_S,), jnp.int32),
            pltpu.VMEM((G_CH, HR), jnp.int32),
            pltpu.VMEM((G_CH, HR), jnp.int32),
            pltpu.VMEM_SHARED((T, HR), jnp.int32),
            pltpu.SemaphoreType.DMA,
            pltpu.SemaphoreType.DMA,
            pltpu.SemaphoreType.DMA,
            pltpu.SemaphoreType.DMA,
        ],
    )
    def _sc_gather(x_hbm, tok_hbm, xs_hbm, tok_v, rows0, rows1, xsh,
                   gs0, gs1, ws0, ws1):
        sid = lax.axis_index("s")
        cid = lax.axis_index("c")
        base = sid * SLOTS_S
        pltpu.sync_copy(tok_hbm.at[pl.ds(base, SLOTS_S)], tok_v)
        rows = (rows0, rows1)
        gsem = (gs0, gs1)
        wsem = (ws0, ws1)
        for r in range(2):
            coff = cid * HQ + r * HR
            if r > 0:
                plsc.subcore_barrier()

            @pl.when(sid == 0)
            def _(coff=coff):
                pltpu.sync_copy(x_hbm.at[:, pl.ds(coff, HR)], xsh)

            plsc.subcore_barrier()
            gc = [None] * G_NCH
            wc = [None] * G_NCH
            gc[0] = pltpu.async_copy(
                xsh.at[tok_v.at[pl.ds(0, G_CH)]], rows[0], gsem[0])
            for c in range(G_NCH):
                b = c % 2
                nb = (c + 1) % 2
                if c + 1 < G_NCH:
                    if c >= 1:
                        wc[c - 1].wait()
                    gc[c + 1] = pltpu.async_copy(
                        xsh.at[tok_v.at[pl.ds((c + 1) * G_CH, G_CH)]],
                        rows[nb], gsem[nb])
                gc[c].wait()
                wc[c] = pltpu.async_copy(
                    rows[b],
                    xs_hbm.at[pl.ds(base + c * G_CH, G_CH),
                              pl.ds(coff, HR)],
                    wsem[b])
            for c in range(max(G_NCH - 2, 0), G_NCH):
                wc[c].wait()

    return _sc_gather


@functools.cache
def _make_sc_combine():
    mesh = plsc.VectorSubcoreMesh(core_axis_name="c", subcore_axis_name="s")

    @functools.partial(
        pl.kernel,
        mesh=mesh,
        out_type=jax.ShapeDtypeStruct((T, H), jnp.float32),
        scratch_types=[
            pltpu.VMEM((2 * T_CH,), jnp.int32),
            pltpu.VMEM((2 * T_CH, H), jnp.float32),
            pltpu.VMEM((T_CH, H), jnp.float32),
            pltpu.SemaphoreType.DMA,
        ],
    )
    def _sc_combine(ysw_hbm, pos_hbm, out_hbm, idx_v, rows_v, out_v, sem):
        wid = lax.axis_index("s") * NC + lax.axis_index("c")
        for c in range(TOK_W // T_CH):
            tbase = wid * TOK_W + c * T_CH
            pltpu.sync_copy(pos_hbm.at[pl.ds(2 * tbase, 2 * T_CH)], idx_v)
            pltpu.async_copy(ysw_hbm.at[idx_v], rows_v, sem).wait()

            def body(i, carry):
                for v in range(H // L):
                    sl = pl.ds(v * L, L)
                    out_v[i, sl] = rows_v[2 * i, sl] + rows_v[2 * i + 1, sl]
                return carry

            lax.fori_loop(0, T_CH, body, 0)
            pltpu.sync_copy(out_v, out_hbm.at[pl.ds(tbase, T_CH)])

    return _sc_combine


def kernel(x, gate_tensor, gate_w, up_w, down_w):
    gt_pad = jnp.pad(gate_tensor, ((0, 0), (0, EPAD - E)))
    iw, ww = pl.pallas_call(
        _router_body,
        out_shape=(
            jax.ShapeDtypeStruct((T, EPAD), jnp.int32),
            jax.ShapeDtypeStruct((T, EPAD), jnp.float32),
        ),
    )(x, gt_pad)
    topk_idx = iw[:, :K]
    flat_w = ww[:, :K].reshape(-1)

    flat_e = topk_idx.reshape(-1)
    onehot = (flat_e[:, None] == jnp.arange(E, dtype=jnp.int32)[None, :]).astype(jnp.int32)
    csum = jnp.cumsum(onehot, axis=0)
    rank = jnp.take_along_axis(csum, flat_e[:, None], axis=1)[:, 0] - 1
    counts = csum[-1]
    nblk = (counts + TB - 1) // TB
    blk_start = jnp.concatenate([jnp.zeros(1, jnp.int32), jnp.cumsum(nblk)[:-1].astype(jnp.int32)])
    tot_blocks = jnp.sum(nblk)
    slot = blk_start[flat_e] * TB + rank
    tok_of_slot = jnp.zeros(NP, jnp.int32).at[slot].set(
        jnp.arange(P, dtype=jnp.int32) // K)
    sortw = jnp.zeros(NP, jnp.float32).at[slot].set(flat_w)
    barange = jnp.arange(NB, dtype=jnp.int32)
    bexp = jnp.sum((barange[:, None] >= blk_start[None, :]).astype(jnp.int32), axis=1) - 1
    bval = (barange < tot_blocks).astype(jnp.int32)

    xu = lax.bitcast_convert_type(x.astype(jnp.bfloat16), jnp.uint16)
    xu = xu.astype(jnp.uint32)
    x_pk = lax.bitcast_convert_type(
        xu[:, :HW] | (xu[:, HW:] << 16), jnp.int32)
    xs_pk = _make_sc_gather()(x_pk, tok_of_slot)

    grid_spec = pltpu.PrefetchScalarGridSpec(
        num_scalar_prefetch=2,
        grid=(NB,),
        in_specs=[
            pl.BlockSpec((TB, HW), lambda b, be, bv: (b, 0)),
            pl.BlockSpec((1, H, F), lambda b, be, bv: (be[b], 0, 0)),
            pl.BlockSpec((1, H, F), lambda b, be, bv: (be[b], 0, 0)),
            pl.BlockSpec((1, F, H), lambda b, be, bv: (be[b], 0, 0)),
            pl.BlockSpec((1, 1, TB), lambda b, be, bv: (b, 0, 0)),
        ],
        out_specs=pl.BlockSpec((TB, H), lambda b, be, bv: (b, 0)),
    )
    ysw = pl.pallas_call(
        _expert_body,
        grid_spec=grid_spec,
        out_shape=jax.ShapeDtypeStruct((NP, H), jnp.float32),
    )(bexp, bval, xs_pk, gate_w, up_w, down_w, sortw.reshape(NB, 1, TB))

    return _make_sc_combine()(ysw, slot)


# speedup vs baseline: 1.2238x; 1.0517x over previous
import functools

import jax
import jax.numpy as jnp
from jax import lax
from jax.experimental import pallas as pl
from jax.experimental.pallas import tpu as pltpu
from jax.experimental.pallas import tpu_sc as plsc

T, H, F, E, K = 2048, 1024, 512, 8, 2
P = T * K
EPAD = 128
TB = 256
NB = P // TB + E
NP = NB * TB

NC, NS, L = 2, 16, 16
NW = NC * NS

HW = H // 2
HQ = HW // NC
HR = HQ // 2
SLOTS_S = NP // NS
G_CH = 96
G_NCH = SLOTS_S // G_CH
TOK_W = T // NW
T_CH = 32


def _router_body(x_ref, gt_ref, iw_ref, ww_ref):
    logits = jnp.dot(x_ref[...], gt_ref[...], preferred_element_type=jnp.float32)
    col = lax.broadcasted_iota(jnp.int32, (T, EPAD), 1)
    valid = col < E
    lg = jnp.where(valid, logits, -1e30)
    m = jnp.max(lg, axis=1, keepdims=True)
    z = jnp.exp(lg - m)
    prob = z / jnp.sum(z, axis=1, keepdims=True)
    prob = jnp.where(valid, prob, -1.0)
    m1 = jnp.max(prob, axis=1, keepdims=True)
    i1 = jnp.min(jnp.where(prob == m1, col, EPAD), axis=1, keepdims=True)
    p2 = jnp.where(col == i1, -1.0, prob)
    m2 = jnp.max(p2, axis=1, keepdims=True)
    i2 = jnp.min(jnp.where(p2 == m2, col, EPAD), axis=1, keepdims=True)
    denom = m1 + m2 + 1e-20
    w1 = m1 / denom
    w2 = m2 / denom
    iw_ref[...] = jnp.where(col == 0, i1, jnp.where(col == 1, i2, 0))
    ww_ref[...] = jnp.where(col == 0, w1, jnp.where(col == 1, w2, 0.0))


def _expert_body(bexp_ref, bval_ref, xs_ref, gw_ref, uw_ref, dw_ref, sw_ref, out_ref):
    b = pl.program_id(0)

    @pl.when(bval_ref[b] == 1)
    def _():
        xi = xs_ref[...]
        lo = lax.bitcast_convert_type(
            (xi & 0xFFFF).astype(jnp.uint16), jnp.bfloat16)
        hi = lax.bitcast_convert_type(
            lax.shift_right_logical(xi, 16).astype(jnp.uint16),
            jnp.bfloat16)
        gw = gw_ref[0].astype(jnp.bfloat16)
        uw = uw_ref[0].astype(jnp.bfloat16)
        g = (jnp.dot(lo, gw[:HW], preferred_element_type=jnp.float32)
             + jnp.dot(hi, gw[HW:], preferred_element_type=jnp.float32))
        u = (jnp.dot(lo, uw[:HW], preferred_element_type=jnp.float32)
             + jnp.dot(hi, uw[HW:], preferred_element_type=jnp.float32))
        a = (g / (1.0 + jnp.exp(-g))) * u
        o = jnp.dot(a.astype(jnp.bfloat16), dw_ref[0].astype(jnp.bfloat16),
                    preferred_element_type=jnp.float32)
        ob = (o * sw_ref[0, 0, :][:, None]).astype(jnp.bfloat16)
        lo_b = lax.bitcast_convert_type(ob[:, :HW], jnp.uint16)
        hi_b = lax.bitcast_convert_type(ob[:, HW:], jnp.uint16)
        out_ref[...] = lax.bitcast_convert_type(
            lo_b.astype(jnp.uint32) | (hi_b.astype(jnp.uint32) << 16),
            jnp.int32)


def _combine_body(yg_ref, out_ref):
    y = yg_ref[...]

    def unpk(w):
        lo = lax.bitcast_convert_type((w & 0xFFFF).astype(jnp.uint16),
                                      jnp.bfloat16)
        hi = lax.bitcast_convert_type(
            lax.shift_right_logical(w, 16).astype(jnp.uint16), jnp.bfloat16)
        return lo, hi

    la, ha = unpk(y[:, :HW])
    lb, hb = unpk(y[:, HW:])
    lo = la.astype(jnp.float32) + lb.astype(jnp.float32)
    hi = ha.astype(jnp.float32) + hb.astype(jnp.float32)
    out_ref[...] = jnp.concatenate([lo, hi], axis=1)


@functools.cache
def _make_sc_gather():
    mesh = plsc.VectorSubcoreMesh(core_axis_name="c", subcore_axis_name="s")

    @functools.partial(
        pl.kernel,
        mesh=mesh,
        out_type=jax.ShapeDtypeStruct((NP, HW), jnp.int32),
        scratch_types=[
            pltpu.VMEM((SLOTS_S,), jnp.int32),
            pltpu.VMEM((G_CH, HR), jnp.int32),
            pltpu.VMEM((G_CH, HR), jnp.int32),
            pltpu.VMEM_SHARED((T, HR), jnp.int32),
            pltpu.SemaphoreType.DMA,
            pltpu.SemaphoreType.DMA,
            pltpu.SemaphoreType.DMA,
            pltpu.SemaphoreType.DMA,
        ],
    )
    def _sc_gather(x_hbm, tok_hbm, xs_hbm, tok_v, rows0, rows1, xsh,
                   gs0, gs1, ws0, ws1):
        sid = lax.axis_index("s")
        cid = lax.axis_index("c")
        base = sid * SLOTS_S
        pltpu.sync_copy(tok_hbm.at[pl.ds(base, SLOTS_S)], tok_v)
        rows = (rows0, rows1)
        gsem = (gs0, gs1)
        wsem = (ws0, ws1)
        for r in range(2):
            coff = cid * HQ + r * HR
            if r > 0:
                plsc.subcore_barrier()

            @pl.when(sid == 0)
            def _(coff=coff):
                pltpu.sync_copy(x_hbm.at[:, pl.ds(coff, HR)], xsh)

            plsc.subcore_barrier()
            gc = [None] * G_NCH
            wc = [None] * G_NCH
            gc[0] = pltpu.async_copy(
                xsh.at[tok_v.at[pl.ds(0, G_CH)]], rows[0], gsem[0])
            for c in range(G_NCH):
                b = c % 2
                nb = (c + 1) % 2
                if c + 1 < G_NCH:
                    if c >= 1:
                        wc[c - 1].wait()
                    gc[c + 1] = pltpu.async_copy(
                        xsh.at[tok_v.at[pl.ds((c + 1) * G_CH, G_CH)]],
                        rows[nb], gsem[nb])
                gc[c].wait()
                wc[c] = pltpu.async_copy(
                    rows[b],
                    xs_hbm.at[pl.ds(base + c * G_CH, G_CH),
                              pl.ds(coff, HR)],
                    wsem[b])
            for c in range(max(G_NCH - 2, 0), G_NCH):
                wc[c].wait()

    return _sc_gather


@functools.cache
def _make_sc_permute():
    mesh = plsc.VectorSubcoreMesh(core_axis_name="c", subcore_axis_name="s")
    rows_w = P // NW
    r_ch = rows_w // 2

    @functools.partial(
        pl.kernel,
        mesh=mesh,
        out_type=jax.ShapeDtypeStruct((P, HW), jnp.int32),
        scratch_types=[
            pltpu.VMEM((rows_w,), jnp.int32),
            pltpu.VMEM((r_ch, HW), jnp.int32),
            pltpu.VMEM((r_ch, HW), jnp.int32),
            pltpu.SemaphoreType.DMA,
            pltpu.SemaphoreType.DMA,
            pltpu.SemaphoreType.DMA,
            pltpu.SemaphoreType.DMA,
        ],
    )
    def _sc_permute(ysw_hbm, pos_hbm, yg_hbm, pos_v, rows0, rows1,
                    gs0, gs1, ws0, ws1):
        wid = lax.axis_index("s") * NC + lax.axis_index("c")
        base = wid * rows_w
        pltpu.sync_copy(pos_hbm.at[pl.ds(base, rows_w)], pos_v)
        rows = (rows0, rows1)
        gsem = (gs0, gs1)
        wsem = (ws0, ws1)
        g0 = pltpu.async_copy(
            ysw_hbm.at[pos_v.at[pl.ds(0, r_ch)]], rows[0], gsem[0])
        g1 = pltpu.async_copy(
            ysw_hbm.at[pos_v.at[pl.ds(r_ch, r_ch)]], rows[1], gsem[1])
        g0.wait()
        w0 = pltpu.async_copy(
            rows[0], yg_hbm.at[pl.ds(base, r_ch)], wsem[0])
        g1.wait()
        w1 = pltpu.async_copy(
            rows[1], yg_hbm.at[pl.ds(base + r_ch, r_ch)], wsem[1])
        w0.wait()
        w1.wait()

    return _sc_permute


def kernel(x, gate_tensor, gate_w, up_w, down_w):
    gt_pad = jnp.pad(gate_tensor, ((0, 0), (0, EPAD - E)))
    iw, ww = pl.pallas_call(
        _router_body,
        out_shape=(
            jax.ShapeDtypeStruct((T, EPAD), jnp.int32),
            jax.ShapeDtypeStruct((T, EPAD), jnp.float32),
        ),
    )(x, gt_pad)
    topk_idx = iw[:, :K]
    flat_w = ww[:, :K].reshape(-1)

    flat_e = topk_idx.reshape(-1)
    onehot = (flat_e[:, None] == jnp.arange(E, dtype=jnp.int32)[None, :]).astype(jnp.int32)
    csum = jnp.cumsum(onehot, axis=0)
    rank = jnp.take_along_axis(csum, flat_e[:, None], axis=1)[:, 0] - 1
    counts = csum[-1]
    nblk = (counts + TB - 1) // TB
    blk_start = jnp.concatenate([jnp.zeros(1, jnp.int32), jnp.cumsum(nblk)[:-1].astype(jnp.int32)])
    tot_blocks = jnp.sum(nblk)
    slot = blk_start[flat_e] * TB + rank
    tok_of_slot = jnp.zeros(NP, jnp.int32).at[slot].set(
        jnp.arange(P, dtype=jnp.int32) // K)
    sortw = jnp.zeros(NP, jnp.float32).at[slot].set(flat_w)
    barange = jnp.arange(NB, dtype=jnp.int32)
    bexp = jnp.sum((barange[:, None] >= blk_start[None, :]).astype(jnp.int32), axis=1) - 1
    bval = (barange < tot_blocks).astype(jnp.int32)

    xu = lax.bitcast_convert_type(x.astype(jnp.bfloat16), jnp.uint16)
    xu = xu.astype(jnp.uint32)
    x_pk = lax.bitcast_convert_type(
        xu[:, :HW] | (xu[:, HW:] << 16), jnp.int32)
    xs_pk = _make_sc_gather()(x_pk, tok_of_slot)

    grid_spec = pltpu.PrefetchScalarGridSpec(
        num_scalar_prefetch=2,
        grid=(NB,),
        in_specs=[
            pl.BlockSpec((TB, HW), lambda b, be, bv: (b, 0)),
            pl.BlockSpec((1, H, F), lambda b, be, bv: (be[b], 0, 0)),
            pl.BlockSpec((1, H, F), lambda b, be, bv: (be[b], 0, 0)),
            pl.BlockSpec((1, F, H), lambda b, be, bv: (be[b], 0, 0)),
            pl.BlockSpec((1, 1, TB), lambda b, be, bv: (b, 0, 0)),
        ],
        out_specs=pl.BlockSpec((TB, HW), lambda b, be, bv: (b, 0)),
    )
    ysw = pl.pallas_call(
        _expert_body,
        grid_spec=grid_spec,
        out_shape=jax.ShapeDtypeStruct((NP, HW), jnp.int32),
    )(bexp, bval, xs_pk, gate_w, up_w, down_w, sortw.reshape(NB, 1, TB))

    yg = _make_sc_permute()(ysw, slot)
    CB = 256
    return pl.pallas_call(
        _combine_body,
        grid=(T // CB,),
        in_specs=[pl.BlockSpec((CB, K * HW), lambda b: (b, 0))],
        out_specs=pl.BlockSpec((CB, H), lambda b: (b, 0)),
        out_shape=jax.ShapeDtypeStruct((T, H), jnp.float32),
    )(yg.reshape(T, K * HW))


# speedup vs baseline: 1.2263x; 1.0020x over previous
import functools

import jax
import jax.numpy as jnp
from jax import lax
from jax.experimental import pallas as pl
from jax.experimental.pallas import tpu as pltpu
from jax.experimental.pallas import tpu_sc as plsc

T, H, F, E, K = 2048, 1024, 512, 8, 2
P = T * K
EPAD = 128
TB = 256
NB = P // TB + E
NP = NB * TB

NC, NS, L = 2, 16, 16
NW = NC * NS

HW = H // 2
HQ = HW // NC
HR = HQ // 2
SLOTS_S = NP // NS
G_CH = 48
G_NCH = SLOTS_S // G_CH
TOK_W = T // NW
T_CH = 32


def _router_body(x_ref, gt_ref, iw_ref, ww_ref):
    logits = jnp.dot(x_ref[...], gt_ref[...], preferred_element_type=jnp.float32)
    col = lax.broadcasted_iota(jnp.int32, (T, EPAD), 1)
    valid = col < E
    lg = jnp.where(valid, logits, -1e30)
    m = jnp.max(lg, axis=1, keepdims=True)
    z = jnp.exp(lg - m)
    prob = z / jnp.sum(z, axis=1, keepdims=True)
    prob = jnp.where(valid, prob, -1.0)
    m1 = jnp.max(prob, axis=1, keepdims=True)
    i1 = jnp.min(jnp.where(prob == m1, col, EPAD), axis=1, keepdims=True)
    p2 = jnp.where(col == i1, -1.0, prob)
    m2 = jnp.max(p2, axis=1, keepdims=True)
    i2 = jnp.min(jnp.where(p2 == m2, col, EPAD), axis=1, keepdims=True)
    denom = m1 + m2 + 1e-20
    w1 = m1 / denom
    w2 = m2 / denom
    iw_ref[...] = jnp.where(col == 0, i1, jnp.where(col == 1, i2, 0))
    ww_ref[...] = jnp.where(col == 0, w1, jnp.where(col == 1, w2, 0.0))


def _expert_body(bexp_ref, bval_ref, xs_ref, gw_ref, uw_ref, dw_ref, sw_ref, out_ref):
    b = pl.program_id(0)

    @pl.when(bval_ref[b] == 1)
    def _():
        xi = xs_ref[...]
        lo = lax.bitcast_convert_type(
            (xi & 0xFFFF).astype(jnp.uint16), jnp.bfloat16)
        hi = lax.bitcast_convert_type(
            lax.shift_right_logical(xi, 16).astype(jnp.uint16),
            jnp.bfloat16)
        gw = gw_ref[0].astype(jnp.bfloat16)
        uw = uw_ref[0].astype(jnp.bfloat16)
        g = (jnp.dot(lo, gw[:HW], preferred_element_type=jnp.float32)
             + jnp.dot(hi, gw[HW:], preferred_element_type=jnp.float32))
        u = (jnp.dot(lo, uw[:HW], preferred_element_type=jnp.float32)
             + jnp.dot(hi, uw[HW:], preferred_element_type=jnp.float32))
        a = (g / (1.0 + jnp.exp(-g))) * u
        o = jnp.dot(a.astype(jnp.bfloat16), dw_ref[0].astype(jnp.bfloat16),
                    preferred_element_type=jnp.float32)
        ob = (o * sw_ref[0, 0, :][:, None]).astype(jnp.bfloat16)
        lo_b = lax.bitcast_convert_type(ob[:, :HW], jnp.uint16)
        hi_b = lax.bitcast_convert_type(ob[:, HW:], jnp.uint16)
        out_ref[...] = lax.bitcast_convert_type(
            lo_b.astype(jnp.uint32) | (hi_b.astype(jnp.uint32) << 16),
            jnp.int32)


def _combine_body(yg_ref, out_ref):
    y = yg_ref[...]

    def unpk(w):
        lo = lax.bitcast_convert_type((w & 0xFFFF).astype(jnp.uint16),
                                      jnp.bfloat16)
        hi = lax.bitcast_convert_type(
            lax.shift_right_logical(w, 16).astype(jnp.uint16), jnp.bfloat16)
        return lo, hi

    la, ha = unpk(y[:, :HW])
    lb, hb = unpk(y[:, HW:])
    lo = la.astype(jnp.float32) + lb.astype(jnp.float32)
    hi = ha.astype(jnp.float32) + hb.astype(jnp.float32)
    out_ref[...] = jnp.concatenate([lo, hi], axis=1)


@functools.cache
def _make_sc_gather():
    mesh = plsc.VectorSubcoreMesh(core_axis_name="c", subcore_axis_name="s")

    @functools.partial(
        pl.kernel,
        mesh=mesh,
        out_type=jax.ShapeDtypeStruct((NP, HW), jnp.int32),
        scratch_types=[
            pltpu.VMEM((SLOTS_S,), jnp.int32),
            pltpu.VMEM((G_CH, HR), jnp.int32),
            pltpu.VMEM((G_CH, HR), jnp.int32),
            pltpu.VMEM_SHARED((T, HR), jnp.int32),
            pltpu.SemaphoreType.DMA,
            pltpu.SemaphoreType.DMA,
            pltpu.SemaphoreType.DMA,
            pltpu.SemaphoreType.DMA,
        ],
    )
    def _sc_gather(x_hbm, tok_hbm, xs_hbm, tok_v, rows0, rows1, xsh,
                   gs0, gs1, ws0, ws1):
        sid = lax.axis_index("s")
        cid = lax.axis_index("c")
        base = sid * SLOTS_S
        pltpu.sync_copy(tok_hbm.at[pl.ds(base, SLOTS_S)], tok_v)
        rows = (rows0, rows1)
        gsem = (gs0, gs1)
        wsem = (ws0, ws1)
        for r in range(2):
            coff = cid * HQ + r * HR
            if r > 0:
                plsc.subcore_barrier()

            @pl.when(sid == 0)
            def _(coff=coff):
                pltpu.sync_copy(x_hbm.at[:, pl.ds(coff, HR)], xsh)

            plsc.subcore_barrier()
            gc = [None] * G_NCH
            wc = [None] * G_NCH
            gc[0] = pltpu.async_copy(
                xsh.at[tok_v.at[pl.ds(0, G_CH)]], rows[0], gsem[0])
            for c in range(G_NCH):
                b = c % 2
                nb = (c + 1) % 2
                if c + 1 < G_NCH:
                    if c >= 1:
                        wc[c - 1].wait()
                    gc[c + 1] = pltpu.async_copy(
                        xsh.at[tok_v.at[pl.ds((c + 1) * G_CH, G_CH)]],
                        rows[nb], gsem[nb])
                gc[c].wait()
                wc[c] = pltpu.async_copy(
                    rows[b],
                    xs_hbm.at[pl.ds(base + c * G_CH, G_CH),
                              pl.ds(coff, HR)],
                    wsem[b])
            for c in range(max(G_NCH - 2, 0), G_NCH):
                wc[c].wait()

    return _sc_gather


@functools.cache
def _make_sc_permute():
    mesh = plsc.VectorSubcoreMesh(core_axis_name="c", subcore_axis_name="s")
    rows_w = P // NW
    r_ch = rows_w // 2

    @functools.partial(
        pl.kernel,
        mesh=mesh,
        out_type=jax.ShapeDtypeStruct((P, HW), jnp.int32),
        scratch_types=[
            pltpu.VMEM((rows_w,), jnp.int32),
            pltpu.VMEM((r_ch, HW), jnp.int32),
            pltpu.VMEM((r_ch, HW), jnp.int32),
            pltpu.SemaphoreType.DMA,
            pltpu.SemaphoreType.DMA,
            pltpu.SemaphoreType.DMA,
            pltpu.SemaphoreType.DMA,
        ],
    )
    def _sc_permute(ysw_hbm, pos_hbm, yg_hbm, pos_v, rows0, rows1,
                    gs0, gs1, ws0, ws1):
        wid = lax.axis_index("s") * NC + lax.axis_index("c")
        base = wid * rows_w
        pltpu.sync_copy(pos_hbm.at[pl.ds(base, rows_w)], pos_v)
        rows = (rows0, rows1)
        gsem = (gs0, gs1)
        wsem = (ws0, ws1)
        g0 = pltpu.async_copy(
            ysw_hbm.at[pos_v.at[pl.ds(0, r_ch)]], rows[0], gsem[0])
        g1 = pltpu.async_copy(
            ysw_hbm.at[pos_v.at[pl.ds(r_ch, r_ch)]], rows[1], gsem[1])
        g0.wait()
        w0 = pltpu.async_copy(
            rows[0], yg_hbm.at[pl.ds(base, r_ch)], wsem[0])
        g1.wait()
        w1 = pltpu.async_copy(
            rows[1], yg_hbm.at[pl.ds(base + r_ch, r_ch)], wsem[1])
        w0.wait()
        w1.wait()

    return _sc_permute


def kernel(x, gate_tensor, gate_w, up_w, down_w):
    gt_pad = jnp.pad(gate_tensor, ((0, 0), (0, EPAD - E)))
    iw, ww = pl.pallas_call(
        _router_body,
        out_shape=(
            jax.ShapeDtypeStruct((T, EPAD), jnp.int32),
            jax.ShapeDtypeStruct((T, EPAD), jnp.float32),
        ),
    )(x, gt_pad)
    topk_idx = iw[:, :K]
    flat_w = ww[:, :K].reshape(-1)

    flat_e = topk_idx.reshape(-1)
    onehot = (flat_e[:, None] == jnp.arange(E, dtype=jnp.int32)[None, :]).astype(jnp.int32)
    csum = jnp.cumsum(onehot, axis=0)
    rank = jnp.take_along_axis(csum, flat_e[:, None], axis=1)[:, 0] - 1
    counts = csum[-1]
    nblk = (counts + TB - 1) // TB
    blk_start = jnp.concatenate([jnp.zeros(1, jnp.int32), jnp.cumsum(nblk)[:-1].astype(jnp.int32)])
    tot_blocks = jnp.sum(nblk)
    slot = blk_start[flat_e] * TB + rank
    tok_of_slot = jnp.zeros(NP, jnp.int32).at[slot].set(
        jnp.arange(P, dtype=jnp.int32) // K)
    sortw = jnp.zeros(NP, jnp.float32).at[slot].set(flat_w)
    barange = jnp.arange(NB, dtype=jnp.int32)
    bexp = jnp.sum((barange[:, None] >= blk_start[None, :]).astype(jnp.int32), axis=1) - 1
    bval = (barange < tot_blocks).astype(jnp.int32)

    xu = lax.bitcast_convert_type(x.astype(jnp.bfloat16), jnp.uint16)
    xu = xu.astype(jnp.uint32)
    x_pk = lax.bitcast_convert_type(
        xu[:, :HW] | (xu[:, HW:] << 16), jnp.int32)
    xs_pk = _make_sc_gather()(x_pk, tok_of_slot)

    grid_spec = pltpu.PrefetchScalarGridSpec(
        num_scalar_prefetch=2,
        grid=(NB,),
        in_specs=[
            pl.BlockSpec((TB, HW), lambda b, be, bv: (b, 0)),
            pl.BlockSpec((1, H, F), lambda b, be, bv: (be[b], 0, 0)),
            pl.BlockSpec((1, H, F), lambda b, be, bv: (be[b], 0, 0)),
            pl.BlockSpec((1, F, H), lambda b, be, bv: (be[b], 0, 0)),
            pl.BlockSpec((1, 1, TB), lambda b, be, bv: (b, 0, 0)),
        ],
        out_specs=pl.BlockSpec((TB, HW), lambda b, be, bv: (b, 0)),
    )
    ysw = pl.pallas_call(
        _expert_body,
        grid_spec=grid_spec,
        out_shape=jax.ShapeDtypeStruct((NP, HW), jnp.int32),
    )(bexp, bval, xs_pk, gate_w, up_w, down_w, sortw.reshape(NB, 1, TB))

    yg = _make_sc_permute()(ysw, slot)
    CB = 256
    return pl.pallas_call(
        _combine_body,
        grid=(T // CB,),
        in_specs=[pl.BlockSpec((CB, K * HW), lambda b: (b, 0))],
        out_specs=pl.BlockSpec((CB, H), lambda b: (b, 0)),
        out_shape=jax.ShapeDtypeStruct((T, H), jnp.float32),
    )(yg.reshape(T, K * HW))
